# baseline (speedup 1.0000x reference)
import functools

import jax
import jax.numpy as jnp
from jax import lax
from jax.experimental import pallas as pl
from jax.experimental.pallas import tpu as pltpu
from jax.experimental.pallas import tpu_sc as plsc

D = 128
TAB = 64
ZERO_ROW = 52

NC = 2
NS = 16
NW = NC * NS

CB = 256


def _prep_body(ids_ref, msk_ref, rank_ref, suit_ref, idx_ref, tab_ref):
    l, cols = ids_ref.shape
    lp = idx_ref.shape[0]
    ids = ids_ref[...]
    m = msk_ref[...]
    sel = ZERO_ROW + m * (ids - ZERO_ROW)
    pad = jnp.full((lp - l, cols), ZERO_ROW, jnp.int32)
    idx_ref[...] = jnp.concatenate([sel, pad], axis=0)

    cid = lax.broadcasted_iota(jnp.int32, (TAB, D), 0)
    col = lax.broadcasted_iota(jnp.int32, (TAB, D), 1)
    oh_r = (cid % 13 == col).astype(jnp.float32)
    oh_s = (cid // 13 == col).astype(jnp.float32)
    tab = (jnp.dot(oh_r, rank_ref[...], preferred_element_type=jnp.float32,
                   precision=lax.Precision.HIGHEST)
           + jnp.dot(oh_s, suit_ref[...], preferred_element_type=jnp.float32,
                     precision=lax.Precision.HIGHEST))
    tab_ref[...] = jnp.where(cid < ZERO_ROW, tab, 0.0)


def _prep(ids_t, msk_t, rank_pad, suit_pad):
    l, b = ids_t.shape
    lp = (l + 7) // 8 * 8
    cols = b // NW
    return pl.pallas_call(
        _prep_body,
        grid=(NW,),
        in_specs=[
            pl.BlockSpec((l, cols), lambda i: (0, i)),
            pl.BlockSpec((l, cols), lambda i: (0, i)),
            pl.BlockSpec((D, D), lambda i: (0, 0)),
            pl.BlockSpec((D, D), lambda i: (0, 0)),
        ],
        out_specs=[
            pl.BlockSpec((lp, cols), lambda i: (0, i)),
            pl.BlockSpec((TAB, D), lambda i: (0, 0)),
        ],
        out_shape=[
            jax.ShapeDtypeStruct((lp, b), jnp.int32),
            jax.ShapeDtypeStruct((TAB, D), jnp.float32),
        ],
    )(ids_t, msk_t, rank_pad, suit_pad)


def _make_sc_lookup(b, l, lp):
    bw = b // NW
    ncb = bw // CB
    mesh = plsc.VectorSubcoreMesh(core_axis_name="c", subcore_axis_name="s")

    @functools.partial(
        pl.kernel,
        mesh=mesh,
        out_type=jax.ShapeDtypeStruct((l, b, D), jnp.float32),
        scratch_types=[
            pltpu.VMEM((TAB, D), jnp.float32),
            pltpu.VMEM((lp, bw), jnp.int32),
            pltpu.VMEM((CB, D), jnp.float32),
            pltpu.VMEM((CB, D), jnp.float32),
            pltpu.SemaphoreType.DMA,
            pltpu.SemaphoreType.DMA,
        ],
    )
    def sc_lookup(tab_hbm, idx_hbm, out_hbm, tab_v, idx_v, buf0, buf1, sem0, sem1):
        wid = lax.axis_index("s") * NC + lax.axis_index("c")
        b0 = wid * bw
        pltpu.sync_copy(tab_hbm, tab_v)
        pltpu.sync_copy(idx_hbm.at[pl.ds(0, lp), pl.ds(b0, bw)], idx_v)

        def lbody(li, carry):
            for c in range(ncb):
                buf = buf0 if c % 2 == 0 else buf1
                sem = sem0 if c % 2 == 0 else sem1

                if c < 2:
                    @pl.when(li > 0)
                    def _drain():
                        pltpu.make_async_copy(
                            buf, out_hbm.at[0, pl.ds(0, CB)], sem).wait()
                else:
                    pltpu.make_async_copy(
                        buf, out_hbm.at[0, pl.ds(0, CB)], sem).wait()

                @plsc.parallel_loop(0, CB // 16, 1, unroll=3)
                def _gbody(g):
                    tvec = idx_v[li, pl.ds(c * CB + g * 16, 16)]
                    for k in range(16):
                        t = tvec[k]
                        row = g * 16 + k
                        for j in range(D // 16):
                            sl = pl.ds(j * 16, 16)
                            buf[row, sl] = tab_v[t, sl]
                pltpu.async_copy(buf, out_hbm.at[li, pl.ds(b0 + c * CB, CB)], sem)
            return carry

        lax.fori_loop(0, l, lbody, 0)
        pltpu.make_async_copy(buf0, out_hbm.at[0, pl.ds(0, CB)], sem0).wait()
        pltpu.make_async_copy(buf1, out_hbm.at[0, pl.ds(0, CB)], sem1).wait()

    return sc_lookup


def kernel(card_ids, slot_mask, rank_emb, suit_emb):
    b, l = card_ids.shape
    rank_pad = jnp.zeros((D, D), jnp.float32).at[:13].set(rank_emb)
    suit_pad = jnp.zeros((D, D), jnp.float32).at[:4].set(suit_emb)
    idx_t, table = _prep(card_ids.astype(jnp.int32).T, slot_mask.astype(jnp.int32).T,
                         rank_pad, suit_pad)
    lp = idx_t.shape[0]
    out_lbd = _make_sc_lookup(b, l, lp)(table, idx_t)
    return out_lbd.transpose(1, 0, 2), slot_mask.astype(bool)


# speedup vs baseline: 1.4879x; 1.4879x over previous
import functools

import jax
import jax.numpy as jnp
from jax import lax
from jax.experimental import pallas as pl
from jax.experimental.pallas import tpu as pltpu
from jax.experimental.pallas import tpu_sc as plsc

D = 128
TAB = 64
ZERO_ROW = 52

NC = 2
NS = 16
NW = NC * NS

CB = 256


def _prep_body(ids_ref, msk_ref, rank_ref, suit_ref, idx_ref, tab_ref):
    l, cols = ids_ref.shape
    lp = idx_ref.shape[0]
    ids = ids_ref[...]
    m = msk_ref[...]
    sel = ZERO_ROW + m * (ids - ZERO_ROW)
    pad = jnp.full((lp - l, cols), ZERO_ROW, jnp.int32)
    idx_ref[...] = jnp.concatenate([sel, pad], axis=0)

    cid = lax.broadcasted_iota(jnp.int32, (TAB, D), 0)
    col = lax.broadcasted_iota(jnp.int32, (TAB, D), 1)
    oh_r = (cid % 13 == col).astype(jnp.float32)
    oh_s = (cid // 13 == col).astype(jnp.float32)
    tab = (jnp.dot(oh_r, rank_ref[...], preferred_element_type=jnp.float32,
                   precision=lax.Precision.HIGHEST)
           + jnp.dot(oh_s, suit_ref[...], preferred_element_type=jnp.float32,
                     precision=lax.Precision.HIGHEST))
    tab_ref[...] = jnp.where(cid < ZERO_ROW, tab, 0.0)


def _prep(ids_t, msk_t, rank_pad, suit_pad):
    l, b = ids_t.shape
    lp = (l + 7) // 8 * 8
    cols = b // NW
    return pl.pallas_call(
        _prep_body,
        grid=(NW,),
        in_specs=[
            pl.BlockSpec((l, cols), lambda i: (0, i)),
            pl.BlockSpec((l, cols), lambda i: (0, i)),
            pl.BlockSpec((D, D), lambda i: (0, 0)),
            pl.BlockSpec((D, D), lambda i: (0, 0)),
        ],
        out_specs=[
            pl.BlockSpec((lp, cols), lambda i: (0, i)),
            pl.BlockSpec((TAB, D), lambda i: (0, 0)),
        ],
        out_shape=[
            jax.ShapeDtypeStruct((lp, b), jnp.int32),
            jax.ShapeDtypeStruct((TAB, D), jnp.float32),
        ],
    )(ids_t, msk_t, rank_pad, suit_pad)


def _make_sc_lookup(b, l, lp):
    bw = b // NW
    ncb = bw // CB
    mesh = plsc.VectorSubcoreMesh(core_axis_name="c", subcore_axis_name="s")

    @functools.partial(
        pl.kernel,
        mesh=mesh,
        out_type=jax.ShapeDtypeStruct((l, b, D), jnp.float32),
        scratch_types=[
            pltpu.VMEM((TAB, D), jnp.float32),
            pltpu.VMEM((lp, bw), jnp.int32),
            pltpu.VMEM((CB, D), jnp.float32),
            pltpu.VMEM((CB, D), jnp.float32),
            pltpu.SemaphoreType.DMA,
            pltpu.SemaphoreType.DMA,
        ],
    )
    def sc_lookup(tab_hbm, idx_hbm, out_hbm, tab_v, idx_v, buf0, buf1, sem0, sem1):
        wid = lax.axis_index("s") * NC + lax.axis_index("c")
        b0 = wid * bw
        pltpu.sync_copy(tab_hbm, tab_v)
        pltpu.sync_copy(idx_hbm.at[pl.ds(0, lp), pl.ds(b0, bw)], idx_v)

        def lbody(li, carry):
            for c in range(ncb):
                buf = buf0 if c % 2 == 0 else buf1
                sem = sem0 if c % 2 == 0 else sem1

                if c < 2:
                    @pl.when(li > 0)
                    def _drain():
                        pltpu.make_async_copy(
                            buf, out_hbm.at[0, pl.ds(0, CB)], sem).wait()
                else:
                    pltpu.make_async_copy(
                        buf, out_hbm.at[0, pl.ds(0, CB)], sem).wait()

                @plsc.parallel_loop(0, CB // 16, 1, unroll=2)
                def _gbody(g):
                    tvec = idx_v[li, pl.ds(c * CB + g * 16, 16)]
                    for k in range(16):
                        t = tvec[k]
                        row = g * 16 + k
                        for j in range(D // 16):
                            sl = pl.ds(j * 16, 16)
                            buf[row, sl] = tab_v[t, sl]
                pltpu.async_copy(buf, out_hbm.at[li, pl.ds(b0 + c * CB, CB)], sem)
            return carry

        lax.fori_loop(0, l, lbody, 0)
        pltpu.make_async_copy(buf0, out_hbm.at[0, pl.ds(0, CB)], sem0).wait()
        pltpu.make_async_copy(buf1, out_hbm.at[0, pl.ds(0, CB)], sem1).wait()

    return sc_lookup


def kernel(card_ids, slot_mask, rank_emb, suit_emb):
    b, l = card_ids.shape
    rank_pad = jnp.zeros((D, D), jnp.float32).at[:13].set(rank_emb)
    suit_pad = jnp.zeros((D, D), jnp.float32).at[:4].set(suit_emb)
    idx_t, table = _prep(card_ids.astype(jnp.int32).T, slot_mask.astype(jnp.int32).T,
                         rank_pad, suit_pad)
    lp = idx_t.shape[0]
    out_lbd = _make_sc_lookup(b, l, lp)(table, idx_t)
    return out_lbd.transpose(1, 0, 2), slot_mask.astype(bool)


# speedup vs baseline: 1.4909x; 1.0020x over previous
import functools

import jax
import jax.numpy as jnp
from jax import lax
from jax.experimental import pallas as pl
from jax.experimental.pallas import tpu as pltpu
from jax.experimental.pallas import tpu_sc as plsc

D = 128
TAB = 64
ZERO_ROW = 52

NC = 2
NS = 16
NW = NC * NS

CB = 256


def _prep_body(ids_ref, msk_ref, rank_ref, suit_ref, idx_ref, tab_ref):
    l, cols = ids_ref.shape
    lp = idx_ref.shape[0]
    ids = ids_ref[...]
    m = msk_ref[...]
    sel = ZERO_ROW + m * (ids - ZERO_ROW)
    pad = jnp.full((lp - l, cols), ZERO_ROW, jnp.int32)
    idx_ref[...] = jnp.concatenate([sel, pad], axis=0)

    cid = lax.broadcasted_iota(jnp.int32, (TAB, D), 0)
    col = lax.broadcasted_iota(jnp.int32, (TAB, D), 1)
    oh_r = (cid % 13 == col).astype(jnp.float32)
    oh_s = (cid // 13 == col).astype(jnp.float32)
    tab = (jnp.dot(oh_r, rank_ref[...], preferred_element_type=jnp.float32,
                   precision=lax.Precision.HIGHEST)
           + jnp.dot(oh_s, suit_ref[...], preferred_element_type=jnp.float32,
                     precision=lax.Precision.HIGHEST))
    tab_ref[...] = jnp.where(cid < ZERO_ROW, tab, 0.0)


def _prep(ids_t, msk_t, rank_pad, suit_pad):
    l, b = ids_t.shape
    lp = (l + 7) // 8 * 8
    cols = b // NW
    return pl.pallas_call(
        _prep_body,
        grid=(NW,),
        in_specs=[
            pl.BlockSpec((l, cols), lambda i: (0, i)),
            pl.BlockSpec((l, cols), lambda i: (0, i)),
            pl.BlockSpec((D, D), lambda i: (0, 0)),
            pl.BlockSpec((D, D), lambda i: (0, 0)),
        ],
        out_specs=[
            pl.BlockSpec((lp, cols), lambda i: (0, i)),
            pl.BlockSpec((TAB, D), lambda i: (0, 0)),
        ],
        out_shape=[
            jax.ShapeDtypeStruct((lp, b), jnp.int32),
            jax.ShapeDtypeStruct((TAB, D), jnp.float32),
        ],
    )(ids_t, msk_t, rank_pad, suit_pad)


def _make_sc_lookup(b, l, lp):
    bw = b // NW
    ncb = bw // CB
    mesh = plsc.VectorSubcoreMesh(core_axis_name="c", subcore_axis_name="s")

    @functools.partial(
        pl.kernel,
        mesh=mesh,
        out_type=jax.ShapeDtypeStruct((l, b, D), jnp.float32),
        scratch_types=[
            pltpu.VMEM((TAB, D), jnp.float32),
            pltpu.VMEM((lp, bw), jnp.int32),
            pltpu.VMEM((CB, D), jnp.float32),
            pltpu.VMEM((CB, D), jnp.float32),
            pltpu.SemaphoreType.DMA,
            pltpu.SemaphoreType.DMA,
        ],
    )
    def sc_lookup(tab_hbm, idx_hbm, out_hbm, tab_v, idx_v, buf0, buf1, sem0, sem1):
        wid = lax.axis_index("s") * NC + lax.axis_index("c")
        b0 = wid * bw
        pltpu.sync_copy(tab_hbm, tab_v)
        pltpu.sync_copy(idx_hbm.at[pl.ds(0, lp), pl.ds(b0, bw)], idx_v)

        def lbody(li, carry):
            for c in range(ncb):
                buf = buf0 if c % 2 == 0 else buf1
                sem = sem0 if c % 2 == 0 else sem1

                if c < 2:
                    @pl.when(li > 0)
                    def _drain():
                        pltpu.make_async_copy(
                            buf, out_hbm.at[0, pl.ds(0, CB)], sem).wait()
                else:
                    pltpu.make_async_copy(
                        buf, out_hbm.at[0, pl.ds(0, CB)], sem).wait()

                @plsc.parallel_loop(0, CB // 16, 1, unroll=2)
                def _gbody(g):
                    tvec = idx_v[li, pl.ds(c * CB + g * 16, 16)]
                    ts = [tvec[k] for k in range(16)]
                    for k in range(16):
                        row = g * 16 + k
                        for j in range(D // 16):
                            sl = pl.ds(j * 16, 16)
                            buf[row, sl] = tab_v[ts[k], sl]
                pltpu.async_copy(buf, out_hbm.at[li, pl.ds(b0 + c * CB, CB)], sem)
            return carry

        lax.fori_loop(0, l, lbody, 0)
        pltpu.make_async_copy(buf0, out_hbm.at[0, pl.ds(0, CB)], sem0).wait()
        pltpu.make_async_copy(buf1, out_hbm.at[0, pl.ds(0, CB)], sem1).wait()

    return sc_lookup


def kernel(card_ids, slot_mask, rank_emb, suit_emb):
    b, l = card_ids.shape
    rank_pad = jnp.zeros((D, D), jnp.float32).at[:13].set(rank_emb)
    suit_pad = jnp.zeros((D, D), jnp.float32).at[:4].set(suit_emb)
    idx_t, table = _prep(card_ids.astype(jnp.int32).T, slot_mask.astype(jnp.int32).T,
                         rank_pad, suit_pad)
    lp = idx_t.shape[0]
    out_lbd = _make_sc_lookup(b, l, lp)(table, idx_t)
    return out_lbd.transpose(1, 0, 2), slot_mask.astype(bool)


# speedup vs baseline: 1.5754x; 1.0567x over previous
import functools

import jax
import jax.numpy as jnp
from jax import lax
from jax.experimental import pallas as pl
from jax.experimental.pallas import tpu as pltpu
from jax.experimental.pallas import tpu_sc as plsc

D = 128
TAB = 64
ZERO_ROW = 52

NC = 2
NS = 16
NW = NC * NS

CB = 256


def _prep_body(ids_ref, msk_ref, rank_ref, suit_ref, idx_ref, tab_ref):
    l, cols = ids_ref.shape
    lp = idx_ref.shape[0]
    ids = ids_ref[...]
    m = msk_ref[...]
    sel = ZERO_ROW + m * (ids - ZERO_ROW)
    pad = jnp.full((lp - l, cols), ZERO_ROW, jnp.int32)
    idx_ref[...] = jnp.concatenate([sel, pad], axis=0)

    @pl.when(pl.program_id(0) == 0)
    def _build_table():
        cid = lax.broadcasted_iota(jnp.int32, (TAB, D), 0)
        col = lax.broadcasted_iota(jnp.int32, (TAB, D), 1)
        oh_r = (cid % 13 == col).astype(jnp.float32)
        oh_s = (cid // 13 == col).astype(jnp.float32)
        tab = (jnp.dot(oh_r, rank_ref[...], preferred_element_type=jnp.float32,
                       precision=lax.Precision.HIGHEST)
               + jnp.dot(oh_s, suit_ref[...], preferred_element_type=jnp.float32,
                         precision=lax.Precision.HIGHEST))
        tab_ref[...] = jnp.where(cid < ZERO_ROW, tab, 0.0)


PREP_GRID = 4


def _prep(ids_t, msk_t, rank_pad, suit_pad):
    l, b = ids_t.shape
    lp = (l + 7) // 8 * 8
    cols = b // PREP_GRID
    return pl.pallas_call(
        _prep_body,
        grid=(PREP_GRID,),
        in_specs=[
            pl.BlockSpec((l, cols), lambda i: (0, i)),
            pl.BlockSpec((l, cols), lambda i: (0, i)),
            pl.BlockSpec((D, D), lambda i: (0, 0)),
            pl.BlockSpec((D, D), lambda i: (0, 0)),
        ],
        out_specs=[
            pl.BlockSpec((lp, cols), lambda i: (0, i)),
            pl.BlockSpec((TAB, D), lambda i: (0, 0)),
        ],
        out_shape=[
            jax.ShapeDtypeStruct((lp, b), jnp.int32),
            jax.ShapeDtypeStruct((TAB, D), jnp.float32),
        ],
    )(ids_t, msk_t, rank_pad, suit_pad)


def _make_sc_lookup(b, l, lp):
    bw = b // NW
    ncb = bw // CB
    mesh = plsc.VectorSubcoreMesh(core_axis_name="c", subcore_axis_name="s")

    @functools.partial(
        pl.kernel,
        mesh=mesh,
        out_type=jax.ShapeDtypeStruct((l, b, D), jnp.float32),
        scratch_types=[
            pltpu.VMEM((TAB, D), jnp.float32),
            pltpu.VMEM((lp, bw), jnp.int32),
            pltpu.VMEM((CB, D), jnp.float32),
            pltpu.VMEM((CB, D), jnp.float32),
            pltpu.SemaphoreType.DMA,
            pltpu.SemaphoreType.DMA,
        ],
    )
    def sc_lookup(tab_hbm, idx_hbm, out_hbm, tab_v, idx_v, buf0, buf1, sem0, sem1):
        wid = lax.axis_index("s") * NC + lax.axis_index("c")
        b0 = wid * bw
        pltpu.sync_copy(tab_hbm, tab_v)
        pltpu.sync_copy(idx_hbm.at[pl.ds(0, lp), pl.ds(b0, bw)], idx_v)

        def lbody(li, carry):
            for c in range(ncb):
                buf = buf0 if c % 2 == 0 else buf1
                sem = sem0 if c % 2 == 0 else sem1

                if c < 2:
                    @pl.when(li > 0)
                    def _drain():
                        pltpu.make_async_copy(
                            buf, out_hbm.at[0, pl.ds(0, CB)], sem).wait()
                else:
                    pltpu.make_async_copy(
                        buf, out_hbm.at[0, pl.ds(0, CB)], sem).wait()

                @plsc.parallel_loop(0, CB // 16, 1, unroll=2)
                def _gbody(g):
                    tvec = idx_v[li, pl.ds(c * CB + g * 16, 16)]
                    ts = [tvec[k] for k in range(16)]
                    for k in range(16):
                        row = g * 16 + k
                        for j in range(D // 16):
                            sl = pl.ds(j * 16, 16)
                            buf[row, sl] = tab_v[ts[k], sl]
                pltpu.async_copy(buf, out_hbm.at[li, pl.ds(b0 + c * CB, CB)], sem)
            return carry

        lax.fori_loop(0, l, lbody, 0)
        pltpu.make_async_copy(buf0, out_hbm.at[0, pl.ds(0, CB)], sem0).wait()
        pltpu.make_async_copy(buf1, out_hbm.at[0, pl.ds(0, CB)], sem1).wait()

    return sc_lookup


def kernel(card_ids, slot_mask, rank_emb, suit_emb):
    b, l = card_ids.shape
    rank_pad = jnp.zeros((D, D), jnp.float32).at[:13].set(rank_emb)
    suit_pad = jnp.zeros((D, D), jnp.float32).at[:4].set(suit_emb)
    idx_t, table = _prep(card_ids.astype(jnp.int32).T, slot_mask.astype(jnp.int32).T,
                         rank_pad, suit_pad)
    lp = idx_t.shape[0]
    out_lbd = _make_sc_lookup(b, l, lp)(table, idx_t)
    return out_lbd.transpose(1, 0, 2), slot_mask.astype(bool)
